# Initial kernel scaffold; baseline (speedup 1.0000x reference)
import jax
import jax.numpy as jnp
from jax.experimental import pallas as pl


def kernel(input_ids, table):
    raise NotImplementedError("write your pallas kernel here")


# speedup vs baseline: 1.4583x; 1.4583x over previous
import functools

import jax
import jax.numpy as jnp
from jax import lax
from jax.experimental import pallas as pl
from jax.experimental.pallas import tpu as pltpu
from jax.experimental.pallas import tpu_sc as plsc

EMBED_DIM = 32
IW = 128
CHUNK = 1024


@functools.partial(jax.jit, static_argnums=(2, 3))
def _sc_gather(idx2d, table, b_total, d):
    info = plsc.get_sparse_core_info()
    nw = info.num_cores * info.num_subcores
    b_per_w = b_total // nw
    n_chunks = b_per_w // CHUNK
    sub = CHUNK // IW
    mesh = plsc.VectorSubcoreMesh(core_axis_name="c", subcore_axis_name="s")

    @functools.partial(
        pl.kernel,
        mesh=mesh,
        out_type=jax.ShapeDtypeStruct((b_total, d), jnp.float32),
        scratch_types=[
            pltpu.VMEM((sub, IW), jnp.int32),
            pltpu.VMEM((CHUNK, d), jnp.float32),
            pltpu.SemaphoreType.DMA,
        ],
        compiler_params=pltpu.CompilerParams(use_tc_tiling_on_sc=False),
    )
    def k(idx_hbm, table_hbm, out_hbm, idx_v, rows_v, sem):
        wid = lax.axis_index("s") * info.num_cores + lax.axis_index("c")
        row_base = wid * (b_per_w // IW)

        def body(i, carry):
            r0 = row_base + i * sub
            pltpu.sync_copy(idx_hbm.at[pl.ds(r0, sub)], idx_v)
            copies = []
            for j in range(sub):
                copies.append(
                    pltpu.async_copy(
                        table_hbm.at[idx_v.at[j]],
                        rows_v.at[pl.ds(j * IW, IW)],
                        sem,
                    )
                )
            for c in copies:
                c.wait()
            pltpu.sync_copy(rows_v, out_hbm.at[pl.ds((r0) * IW, CHUNK)])
            return carry

        lax.fori_loop(0, n_chunks, body, 0)

    return k(idx2d, table)


def kernel(input_ids, table):
    b, s = input_ids.shape
    d = table.shape[1]
    flat = input_ids.reshape(b * s)
    idx2d = flat.reshape(-1, IW).astype(jnp.int32)
    out = _sc_gather(idx2d, table, b * s, d)
    return out.reshape(b, s, d)


# speedup vs baseline: 1.4932x; 1.0239x over previous
import functools

import jax
import jax.numpy as jnp
from jax import lax
from jax.experimental import pallas as pl
from jax.experimental.pallas import tpu as pltpu
from jax.experimental.pallas import tpu_sc as plsc

EMBED_DIM = 32
IW = 128
SUB = 10
CHUNK = SUB * IW


@functools.partial(jax.jit, static_argnums=(2, 3))
def _sc_gather(idx2d, table, b_total, d):
    info = plsc.get_sparse_core_info()
    nw = info.num_cores * info.num_subcores
    b_per_w = b_total // nw
    irows_per_w = b_per_w // IW
    n_chunks = b_per_w // CHUNK
    assert n_chunks % 2 == 0
    mesh = plsc.VectorSubcoreMesh(core_axis_name="c", subcore_axis_name="s")

    @functools.partial(
        pl.kernel,
        mesh=mesh,
        out_type=jax.ShapeDtypeStruct((b_total, d), jnp.float32),
        scratch_types=[
            pltpu.VMEM((irows_per_w, IW), jnp.int32),
            pltpu.VMEM((2, CHUNK, d), jnp.float32),
            pltpu.SemaphoreType.DMA,
            pltpu.SemaphoreType.DMA,
            pltpu.SemaphoreType.DMA,
            pltpu.SemaphoreType.DMA,
        ],
        compiler_params=pltpu.CompilerParams(use_tc_tiling_on_sc=False),
    )
    def k(idx_hbm, table_hbm, out_hbm, idx_v, rows_v, g0, g1, o0, o1):
        wid = lax.axis_index("s") * info.num_cores + lax.axis_index("c")
        irow0 = wid * irows_per_w
        elem0 = wid * b_per_w
        gsem = (g0, g1)
        osem = (o0, o1)

        pltpu.sync_copy(idx_hbm.at[pl.ds(irow0, irows_per_w)], idx_v)

        def gather_fire(i, b):
            for j in range(SUB):
                pltpu.async_copy(
                    table_hbm.at[idx_v.at[i * SUB + j]],
                    rows_v.at[b].at[pl.ds(j * IW, IW)],
                    gsem[b],
                )

        def store_fire(i, b):
            pltpu.async_copy(
                rows_v.at[b],
                out_hbm.at[pl.ds(elem0 + i * CHUNK, CHUNK)],
                osem[b],
            )

        def wait_gather(b):
            pltpu.make_async_copy(
                out_hbm.at[pl.ds(elem0, CHUNK)], rows_v.at[b], gsem[b]
            ).wait()

        def wait_store(b):
            pltpu.make_async_copy(
                rows_v.at[b], out_hbm.at[pl.ds(elem0, CHUNK)], osem[b]
            ).wait()

        def body(g, carry):
            i0 = g * 2
            i1 = i0 + 1

            @pl.when(g > 0)
            def _():
                wait_gather(1)
                store_fire(i0 - 1, 1)
                wait_store(0)

            gather_fire(i0, 0)
            wait_gather(0)
            store_fire(i0, 0)

            @pl.when(g > 0)
            def _():
                wait_store(1)

            gather_fire(i1, 1)
            return carry

        lax.fori_loop(0, n_chunks // 2, body, 0)
        wait_gather(1)
        store_fire(n_chunks - 1, 1)
        wait_store(0)
        wait_store(1)

    return k(idx2d, table)


def kernel(input_ids, table):
    b, s = input_ids.shape
    d = table.shape[1]
    idx2d = input_ids.reshape(-1, IW).astype(jnp.int32)
    out = _sc_gather(idx2d, table, b * s, d)
    return out.reshape(b, s, d)
